# baseline (speedup 1.0000x reference)
import functools

import jax
import jax.numpy as jnp
from jax.experimental import pallas as pl
from jax.experimental.pallas import tpu as pltpu

_ALPHA = 0.2
_EPS = 1e-5


def _hst_body(x_ref, w_ref, b_ref, asrc_ref, adst_ref, h_ref, s_ref, t_ref):
    h = jax.lax.dot_general(
        x_ref[...], w_ref[...], (((1,), (1,)), ((), ())),
        preferred_element_type=jnp.float32) + b_ref[...]
    h_ref[...] = h
    s_ref[...] = jax.lax.dot_general(
        h, asrc_ref[...], (((1,), (0,)), ((), ())),
        preferred_element_type=jnp.float32)
    t_ref[...] = jax.lax.dot_general(
        h, adst_ref[...], (((1,), (0,)), ((), ())),
        preferred_element_type=jnp.float32)


def _gat_body(adj_ref, s_ref, t_ref, h_ref, g_ref, be_ref, o_ref, acc_ref,
              *, n, tk, nk):
    k = pl.program_id(1)

    @pl.when(k == 0)
    def _():
        acc_ref[...] = jnp.zeros_like(acc_ref)

    x = s_ref[...] + t_ref[...]
    neg = -x
    e = jnp.exp(jnp.minimum(neg, _ALPHA * neg))
    w = adj_ref[...] * e
    tm = w.shape[0]
    col = k * tk + jax.lax.broadcasted_iota(jnp.int32, (tm, tk), 1)
    w = jnp.where(col < n, w, 0.0)
    acc_ref[...] += jax.lax.dot_general(
        w, h_ref[...], (((1,), (0,)), ((), ())),
        preferred_element_type=jnp.float32)

    @pl.when(k == nk - 1)
    def _():
        hp = acc_ref[...]
        mean = jnp.mean(hp, axis=1, keepdims=True)
        c = hp - mean
        var = jnp.mean(c * c, axis=1, keepdims=True)
        hn = c * jax.lax.rsqrt(var + _EPS) * g_ref[...] + be_ref[...]
        o_ref[...] = jnp.where(hn > 0, hn, jnp.exp(jnp.minimum(hn, 0.0)) - 1.0)


def kernel(input, adj, W, b, a, gamma, beta):
    n, f = input.shape

    tm2 = 2000 if n % 2000 == 0 else (128 if n % 128 == 0 else 8)
    asrc = a[0, :f].reshape(f, 1)
    adst = a[0, f:].reshape(f, 1)
    h, s, t = pl.pallas_call(
        _hst_body,
        grid=(n // tm2,),
        in_specs=[
            pl.BlockSpec((tm2, f), lambda i: (i, 0)),
            pl.BlockSpec((f, f), lambda i: (0, 0)),
            pl.BlockSpec((1, f), lambda i: (0, 0)),
            pl.BlockSpec((f, 1), lambda i: (0, 0)),
            pl.BlockSpec((f, 1), lambda i: (0, 0)),
        ],
        out_specs=[
            pl.BlockSpec((tm2, f), lambda i: (i, 0)),
            pl.BlockSpec((tm2, 1), lambda i: (i, 0)),
            pl.BlockSpec((tm2, 1), lambda i: (i, 0)),
        ],
        out_shape=[
            jax.ShapeDtypeStruct((n, f), jnp.float32),
            jax.ShapeDtypeStruct((n, 1), jnp.float32),
            jax.ShapeDtypeStruct((n, 1), jnp.float32),
        ],
    )(input, W, b.reshape(1, f), asrc, adst)

    tm = 400 if n % 400 == 0 else (128 if n % 128 == 0 else 8)
    tk = 2048
    nk = pl.cdiv(n, tk)
    npad = nk * tk
    h_pad = jnp.pad(h, ((0, npad - n), (0, 0)))
    t_row = jnp.pad(t, ((0, npad - n), (0, 0))).reshape(1, npad)

    out = pl.pallas_call(
        functools.partial(_gat_body, n=n, tk=tk, nk=nk),
        grid=(n // tm, nk),
        in_specs=[
            pl.BlockSpec((tm, tk), lambda i, k: (i, k)),
            pl.BlockSpec((tm, 1), lambda i, k: (i, 0)),
            pl.BlockSpec((1, tk), lambda i, k: (0, k)),
            pl.BlockSpec((tk, f), lambda i, k: (k, 0)),
            pl.BlockSpec((1, f), lambda i, k: (0, 0)),
            pl.BlockSpec((1, f), lambda i, k: (0, 0)),
        ],
        out_specs=pl.BlockSpec((tm, f), lambda i, k: (i, 0)),
        out_shape=jax.ShapeDtypeStruct((n, f), jnp.float32),
        scratch_shapes=[pltpu.VMEM((tm, f), jnp.float32)],
    )(adj, s, t_row, h_pad, gamma.reshape(1, f), beta.reshape(1, f))
    return out
